# baseline (speedup 1.0000x reference)
import functools
import jax
import jax.numpy as jnp
from jax import lax
from jax.experimental import pallas as pl
from jax.experimental.pallas import tpu as pltpu, tpu_sc as plsc

_CHUNK = 16


def _make_sc(B, S, D):
    info = plsc.get_sparse_core_info()
    NC, NS, L = info.num_cores, info.num_subcores, info.num_lanes
    NW = NC * NS
    s_per_w = S // NW
    n_chunks = s_per_w // _CHUNK
    vregs_per_row = D // L
    n_steps = n_chunks * B
    mesh = plsc.VectorSubcoreMesh(core_axis_name="c", subcore_axis_name="s")

    @functools.partial(
        pl.kernel,
        mesh=mesh,
        out_type=jax.ShapeDtypeStruct((B, S, D), jnp.float32),
        scratch_types=(
            [pltpu.VMEM((_CHUNK, D), jnp.float32)] * 2
            + [pltpu.VMEM((_CHUNK, D), jnp.float32)] * 4
            + [pltpu.SemaphoreType.DMA] * 10
        ),
    )
    def k(x_hbm, pos_hbm, out_hbm,
          pos_v0, pos_v1, x_v0, x_v1, x_v2, x_v3,
          sp0, sp1, sx0, sx1, sx2, sx3, so0, so1, so2, so3):
        wid = lax.axis_index("s") * NC + lax.axis_index("c")
        base = wid * s_per_w
        pos_bufs, x_bufs = [pos_v0, pos_v1], [x_v0, x_v1, x_v2, x_v3]
        sem_p, sem_x = [sp0, sp1], [sx0, sx1, sx2, sx3]
        sem_o = [so0, so1, so2, so3]

        x_handles = [None] * 4
        pos_handles = [None] * 2
        out_handles = [None] * 4

        def issue_x(s):
            t, b = divmod(s, B)
            bi = s % 4
            x_handles[bi] = pltpu.async_copy(
                x_hbm.at[b, pl.ds(base + t * _CHUNK, _CHUNK)],
                x_bufs[bi], sem_x[bi])

        def issue_pos(t):
            pp = t % 2
            pos_handles[pp] = pltpu.async_copy(
                pos_hbm.at[pl.ds(base + t * _CHUNK, _CHUNK)],
                pos_bufs[pp], sem_p[pp])

        issue_pos(0)
        issue_x(0)
        if n_steps > 1:
            issue_x(1)

        for s in range(n_steps):
            t, b = divmod(s, B)
            bi = s % 4
            pp = t % 2

            x_handles[bi].wait()
            if b == 0:
                pos_handles[pp].wait()

            xb, pb = x_bufs[bi], pos_bufs[pp]

            def add_row(r, carry, xb=xb, pb=pb):
                def add_col(c, carry2):
                    plsc.addupdate(xb.at[r, pl.ds(c * L, L)],
                                   pb[r, pl.ds(c * L, L)])
                    return carry2

                lax.fori_loop(0, vregs_per_row, add_col, 0, unroll=8)
                return carry

            lax.fori_loop(0, _CHUNK, add_row, 0)

            out_handles[bi] = pltpu.async_copy(
                xb, out_hbm.at[b, pl.ds(base + t * _CHUNK, _CHUNK)],
                sem_o[bi])

            if b == 0 and t + 1 < n_chunks:
                issue_pos(t + 1)

            u = s + 2
            if u < n_steps:
                if out_handles[u % 4] is not None:
                    out_handles[u % 4].wait()
                    out_handles[u % 4] = None
                issue_x(u)

        for bi in range(4):
            if out_handles[bi] is not None:
                out_handles[bi].wait()

    return k


def kernel(x, pos_table):
    B, S, D = x.shape
    pos = pos_table[:S]
    return _make_sc(B, S, D)(x, pos)


# speedup vs baseline: 2.0116x; 2.0116x over previous
import functools
import jax
import jax.numpy as jnp
from jax import lax
from jax.experimental import pallas as pl
from jax.experimental.pallas import tpu as pltpu, tpu_sc as plsc

_CHUNK = 16


def _make_sc(B, S, D):
    info = plsc.get_sparse_core_info()
    NC, NS, L = info.num_cores, info.num_subcores, info.num_lanes
    NW = NC * NS
    s_per_w = S // NW
    n_chunks = s_per_w // _CHUNK
    vregs_per_row = D // L
    n_steps = n_chunks * B
    mesh = plsc.VectorSubcoreMesh(core_axis_name="c", subcore_axis_name="s")

    @functools.partial(
        pl.kernel,
        mesh=mesh,
        out_type=jax.ShapeDtypeStruct((B, S, D), jnp.float32),
        scratch_types=(
            [pltpu.VMEM((_CHUNK, D), jnp.float32)] * 7
            + [pltpu.SemaphoreType.DMA] * 7
        ),
    )
    def k(x_hbm, pos_hbm, out_hbm,
          pos_v0, pos_v1, x_v0, x_v1, x_v2, o_v0, o_v1,
          sp0, sp1, sx0, sx1, sx2, so0, so1):
        wid = lax.axis_index("s") * NC + lax.axis_index("c")
        base = wid * s_per_w
        pos_bufs = [pos_v0, pos_v1]
        x_bufs = [x_v0, x_v1, x_v2]
        out_bufs = [o_v0, o_v1]
        sem_p, sem_x, sem_o = [sp0, sp1], [sx0, sx1, sx2], [so0, so1]

        x_handles = [None] * 3
        pos_handles = [None] * 2
        out_handles = [None] * 2

        def issue_x(s):
            t, b = divmod(s, B)
            xi = s % 3
            x_handles[xi] = pltpu.async_copy(
                x_hbm.at[b, pl.ds(base + t * _CHUNK, _CHUNK)],
                x_bufs[xi], sem_x[xi])

        def issue_pos(t):
            pp = t % 2
            pos_handles[pp] = pltpu.async_copy(
                pos_hbm.at[pl.ds(base + t * _CHUNK, _CHUNK)],
                pos_bufs[pp], sem_p[pp])

        issue_pos(0)
        for s0 in range(min(3, n_steps)):
            issue_x(s0)

        for s in range(n_steps):
            t, b = divmod(s, B)
            xi, oi, pp = s % 3, s % 2, t % 2

            x_handles[xi].wait()
            if b == 0:
                pos_handles[pp].wait()
            if out_handles[oi] is not None:
                out_handles[oi].wait()
                out_handles[oi] = None

            xb, pb, ob = x_bufs[xi], pos_bufs[pp], out_bufs[oi]

            @plsc.parallel_loop(0, _CHUNK * vregs_per_row, unroll=8)
            def _add(i, xb=xb, pb=pb, ob=ob):
                r = i // vregs_per_row
                sl = pl.ds((i % vregs_per_row) * L, L)
                ob[r, sl] = xb[r, sl] + pb[r, sl]

            out_handles[oi] = pltpu.async_copy(
                ob, out_hbm.at[b, pl.ds(base + t * _CHUNK, _CHUNK)],
                sem_o[oi])

            if b == 0 and t + 1 < n_chunks:
                issue_pos(t + 1)

            if s + 3 < n_steps:
                issue_x(s + 3)

        for oi in range(2):
            if out_handles[oi] is not None:
                out_handles[oi].wait()

    return k


def kernel(x, pos_table):
    B, S, D = x.shape
    pos = pos_table[:S]
    return _make_sc(B, S, D)(x, pos)
